# baseline (speedup 1.0000x reference)
import jax
import jax.numpy as jnp
from jax import lax
from jax.experimental import pallas as pl
from jax.experimental.pallas import tpu as pltpu

EPS = 1e-5


def _spade_nchw_kernel(x_ref, seg_ref, mean_ref, invstd_ref,
                       lcat_ref, rscat_ref, bs_ref, wgb_ref, bgb_ref, o_ref):
    f32 = jnp.float32
    bf16 = jnp.bfloat16
    h = lcat_ref.shape[0]
    wnh = bs_ref.shape[1]
    wc = bgb_ref.shape[1] // 2
    c = x_ref.shape[1]
    hw = x_ref.shape[2]
    w = wc // c

    t_cat = jnp.dot(seg_ref[0], rscat_ref[...], preferred_element_type=f32)
    stacked_t = jnp.concatenate(
        [t_cat[:, 0 * wnh:1 * wnh],
         t_cat[:, 1 * wnh:2 * wnh],
         t_cat[:, 2 * wnh:3 * wnh]], axis=0).astype(bf16)
    acc_a = jnp.dot(lcat_ref[...], stacked_t, preferred_element_type=f32)
    actv = jnp.maximum(acc_a + bs_ref[...], 0.0)

    row = lax.broadcasted_iota(jnp.int32, (h, wnh), 0)
    up = jnp.where(row == 0, 0.0, pltpu.roll(actv, shift=1, axis=0))
    dn = jnp.where(row == h - 1, 0.0, pltpu.roll(actv, shift=h - 1, axis=0))
    shifted_cat = jnp.concatenate([up, actv, dn], axis=1).astype(bf16)
    gb = jnp.dot(shifted_cat, wgb_ref[...], preferred_element_type=f32) + bgb_ref[...]

    gamma_t = pltpu.einshape("h(wc)->c(hw)", gb[:, :wc], w=w)
    beta_t = pltpu.einshape("h(wc)->c(hw)", gb[:, wc:], w=w)

    normalized = (x_ref[0] - mean_ref[...]) * invstd_ref[...]
    o_ref[0] = (normalized * (1.0 + gamma_t) + beta_t).astype(o_ref.dtype)


def kernel(x_nchw, seg_nchw, l_cat, rs_cat, wgb, bias_s, bias_gb):
    n, c, h, w = x_nchw.shape
    _, nc, sh, sw = seg_nchw.shape
    wc = w * c
    swnc = sw * nc
    wnh = bias_s.shape[1]
    hw = h * w

    x = x_nchw.astype(jnp.float32)

    cnt = n * h * w
    s1 = jnp.sum(x, axis=(0, 2, 3))
    s2 = jnp.sum(jnp.square(x), axis=(0, 2, 3))
    mean = s1 / cnt
    var = jnp.maximum(s2 / cnt - jnp.square(mean), 0.0)
    invstd = lax.rsqrt(var + EPS)
    mean_col = mean.reshape(c, 1)
    invstd_col = invstd.reshape(c, 1)

    x_flat = x.reshape(n, c, hw)
    seg_flat = jnp.transpose(seg_nchw, (0, 2, 3, 1)).reshape(n, sh, swnc)
    seg_flat = seg_flat.astype(jnp.bfloat16)

    out_flat = pl.pallas_call(
        _spade_nchw_kernel,
        out_shape=jax.ShapeDtypeStruct((n, c, hw), jnp.float32),
        grid_spec=pltpu.PrefetchScalarGridSpec(
            num_scalar_prefetch=0,
            grid=(n,),
            in_specs=[
                pl.BlockSpec((1, c, hw), lambda i: (i, 0, 0)),
                pl.BlockSpec((1, sh, swnc), lambda i: (i, 0, 0)),
                pl.BlockSpec((c, 1), lambda i: (0, 0)),
                pl.BlockSpec((c, 1), lambda i: (0, 0)),
                pl.BlockSpec((h, 3 * sh), lambda i: (0, 0)),
                pl.BlockSpec((swnc, 3 * wnh), lambda i: (0, 0)),
                pl.BlockSpec((1, wnh), lambda i: (0, 0)),
                pl.BlockSpec((3 * wnh, 2 * wc), lambda i: (0, 0)),
                pl.BlockSpec((1, 2 * wc), lambda i: (0, 0)),
            ],
            out_specs=pl.BlockSpec((1, c, hw), lambda i: (i, 0, 0)),
        ),
        compiler_params=pltpu.CompilerParams(dimension_semantics=("parallel",)),
    )(x_flat, seg_flat, mean_col, invstd_col,
      l_cat, rs_cat, bias_s, wgb, bias_gb)

    return out_flat.reshape(n, c, h, w)


# speedup vs baseline: 1.6332x; 1.6332x over previous
import jax
import jax.numpy as jnp
from jax import lax
from jax.experimental import pallas as pl
from jax.experimental.pallas import tpu as pltpu

EPS = 1e-5


def _spade_kernel(x_ref, seg_ref, mean_ref, invstd_ref,
                  lcatT_ref, rscatT_ref, bsT_ref, wgbT_ref, bgbT_ref, o_ref):
    f32 = jnp.float32
    bf16 = jnp.bfloat16
    h = lcatT_ref.shape[1]
    wnh = bsT_ref.shape[0]
    wc2 = bgbT_ref.shape[0]
    wc = wc2 // 2
    c = x_ref.shape[1]
    hw = x_ref.shape[2]
    w = wc // c

    t_catT = jnp.dot(rscatT_ref[...], seg_ref[0], preferred_element_type=f32)
    stackedT = jnp.concatenate(
        [t_catT[0 * wnh:1 * wnh, :],
         t_catT[1 * wnh:2 * wnh, :],
         t_catT[2 * wnh:3 * wnh, :]], axis=1).astype(bf16)
    accT = jnp.dot(stackedT, lcatT_ref[...], preferred_element_type=f32)
    actvT = jnp.maximum(accT + bsT_ref[...], 0.0)

    col = lax.broadcasted_iota(jnp.int32, (wnh, h), 1)
    up = jnp.where(col == 0, 0.0, pltpu.roll(actvT, shift=1, axis=1))
    dn = jnp.where(col == h - 1, 0.0, pltpu.roll(actvT, shift=h - 1, axis=1))
    shiftedT = jnp.concatenate([up, actvT, dn], axis=0).astype(bf16)
    gbT = jnp.dot(wgbT_ref[...], shiftedT, preferred_element_type=f32) + bgbT_ref[...]

    gamma_t = pltpu.einshape("(cw)h->c(hw)", gbT[:wc, :], c=c)
    beta_t = pltpu.einshape("(cw)h->c(hw)", gbT[wc:, :], c=c)

    normalized = (x_ref[0] - mean_ref[...]) * invstd_ref[...]
    o_ref[0] = (normalized * (1.0 + gamma_t) + beta_t).astype(o_ref.dtype)


def kernel(x_nchw, seg_nchw, l_cat, rs_cat, wgb, bias_s, bias_gb):
    n, c, h, w = x_nchw.shape
    _, nc, sh, sw = seg_nchw.shape
    wc = w * c
    swnc = sw * nc
    wnh = bias_s.shape[1]
    hw = h * w

    x = x_nchw.astype(jnp.float32)

    cnt = n * h * w
    s1 = jnp.sum(x, axis=(0, 2, 3))
    s2 = jnp.sum(jnp.square(x), axis=(0, 2, 3))
    mean = s1 / cnt
    var = jnp.maximum(s2 / cnt - jnp.square(mean), 0.0)
    invstd = lax.rsqrt(var + EPS)
    mean_col = mean.reshape(c, 1)
    invstd_col = invstd.reshape(c, 1)

    l_catT = jnp.transpose(l_cat)
    rs_catT = jnp.transpose(rs_cat)
    perm = (jnp.arange(wc).reshape(w, c).T.reshape(wc))
    wgb_p = jnp.concatenate([wgb[:, perm], wgb[:, wc + perm]], axis=1)
    wgbT = jnp.transpose(wgb_p)
    bgb_p = jnp.concatenate([bias_gb[0, perm], bias_gb[0, wc + perm]])
    bgbT = bgb_p.reshape(2 * wc, 1)
    bsT = jnp.transpose(bias_s)

    x_flat = x.reshape(n, c, hw)
    segT = jnp.transpose(seg_nchw, (0, 3, 1, 2)).reshape(n, swnc, sh)
    segT = segT.astype(jnp.bfloat16)

    out_flat = pl.pallas_call(
        _spade_kernel,
        out_shape=jax.ShapeDtypeStruct((n, c, hw), jnp.float32),
        grid_spec=pltpu.PrefetchScalarGridSpec(
            num_scalar_prefetch=0,
            grid=(n,),
            in_specs=[
                pl.BlockSpec((1, c, hw), lambda i: (i, 0, 0)),
                pl.BlockSpec((1, swnc, sh), lambda i: (i, 0, 0)),
                pl.BlockSpec((c, 1), lambda i: (0, 0)),
                pl.BlockSpec((c, 1), lambda i: (0, 0)),
                pl.BlockSpec((3 * sh, h), lambda i: (0, 0)),
                pl.BlockSpec((3 * wnh, swnc), lambda i: (0, 0)),
                pl.BlockSpec((wnh, 1), lambda i: (0, 0)),
                pl.BlockSpec((2 * wc, 3 * wnh), lambda i: (0, 0)),
                pl.BlockSpec((2 * wc, 1), lambda i: (0, 0)),
            ],
            out_specs=pl.BlockSpec((1, c, hw), lambda i: (i, 0, 0)),
        ),
        compiler_params=pltpu.CompilerParams(dimension_semantics=("parallel",)),
    )(x_flat, segT, mean_col, invstd_col,
      l_catT, rs_catT, bsT, wgbT, bgbT)

    return out_flat.reshape(n, c, h, w)
